# baseline (speedup 1.0000x reference)
import functools

import jax
import jax.numpy as jnp
from jax import lax
from jax.experimental import pallas as pl
from jax.experimental.pallas import tpu as pltpu
from jax.experimental.pallas import tpu_sc as plsc

N_R = 65536
N_B = 128
N_S = 128
L = 16
N_WORKERS = 32
NBP = N_B + 1
C = 2
CL = C * L
G = 4
SZ = G * CL * NBP
NSUP = N_R // N_WORKERS // (G * CL)


def _sc_body(bins_hbm, w_hbm, out_hbm,
             bins0_v, w0_v, out0_v, bins1_v, w1_v, out1_v,
             a_v, b_v, semi0, semi1, semo0, semo1):
    cid = lax.axis_index("c")
    sid = lax.axis_index("s")
    wid = sid * 2 + cid
    rays_per_w = N_R // N_WORKERS

    lane = lax.broadcasted_iota(jnp.int32, (L,), 0)
    laneP = [lane * NBP + k * (L * NBP) for k in range(C)]
    klane = [jnp.full((L,), k * L, jnp.int32) + lane for k in range(C)]
    zero16 = jnp.zeros((L,), jnp.float32)
    one16i = jnp.ones((L,), jnp.int32)
    MAGIC = 12582912.0

    def rbase(s):
        return (wid * rays_per_w + s * (G * CL)) * NBP

    def issue_in(s, bins_v, w_v, sem):
        pltpu.async_copy(bins_hbm.at[pl.ds(rbase(s), SZ)], bins_v, sem)
        pltpu.async_copy(w_hbm.at[pl.ds(rbase(s), SZ)], w_v, sem)

    def wait_in(bins_v, w_v, sem):
        pltpu.make_async_copy(bins_hbm.at[pl.ds(0, SZ)], bins_v, sem).wait()
        pltpu.make_async_copy(w_hbm.at[pl.ds(0, SZ)], w_v, sem).wait()

    def issue_out(s, out_v, sem):
        pltpu.async_copy(out_v, out_hbm.at[pl.ds(rbase(s), SZ)], sem)

    def wait_out(out_v, sem):
        pltpu.make_async_copy(out_v, out_hbm.at[pl.ds(0, SZ)], sem).wait()

    def _init(j, c):
        a_v[pl.ds(j * L, L)] = zero16
        b_v[pl.ds(j * L, L)] = zero16
        return c
    lax.fori_loop(0, N_S * C, _init, 0)

    def compute(bins_v, w_v, out_v):
        def group_body(g, carry2):
            ofs = g * (CL * NBP)
            gP = [laneP[k] + ofs for k in range(C)]

            @plsc.parallel_loop(0, N_B, unroll=4,
                                carry=tuple((zero16, gP[k]) for k in range(C)))
            def tot(i, c):
                out = []
                for k in range(C):
                    acc, idxv = c[k]
                    col = plsc.load_gather(w_v, [idxv])
                    out.append((acc + col, idxv + one16i))
                return tuple(out)

            inv128 = [128.0 / tot[k][0] for k in range(C)]

            def scatter(k, scdf0, dA, dB):
                jf = (scdf0 + MAGIC) - MAGIC
                j = jf.astype(jnp.int32)
                j = jnp.minimum(j, N_S)
                idx = j * CL + klane[k]
                plsc.addupdate_scatter(a_v, [idx], dA)
                plsc.addupdate_scatter(b_v, [idx], dB)

            bins0 = [plsc.load_gather(bins_v, [gP[k]]) for k in range(C)]

            @plsc.parallel_loop(0, N_B, unroll=2,
                                carry=tuple(
                                    (zero16, zero16, bins0[k], zero16, zero16,
                                     gP[k]) for k in range(C)))
            def cfin(i, c):
                out = []
                for k in range(C):
                    acc, scdf0, bi, A_prev, B_prev, idxv = c[k]
                    wcol = plsc.load_gather(w_v, [idxv])
                    acc = acc + wcol
                    scdf1 = acc * inv128[k]
                    idxv1 = idxv + one16i
                    bi1 = plsc.load_gather(bins_v, [idxv1])
                    denom = scdf1 - scdf0
                    denom = jnp.where(denom < 1.28e-3, 128.0, denom)
                    B = (bi1 - bi) / denom
                    A = bi - scdf0 * B
                    scatter(k, scdf0, A - A_prev, B - B_prev)
                    out.append((acc, scdf1, bi1, A, B, idxv1))
                return tuple(out)

            for k in range(C):
                _, scdfN, binsN, A_prev, B_prev, _ = cfin[k]
                scatter(k, scdfN, binsN - A_prev, -B_prev)

            u0 = jnp.full((L,), 0.5, jnp.float32)

            @plsc.parallel_loop(0, N_S, unroll=4,
                                carry=tuple((zero16, zero16, gP[k])
                                            for k in range(C)) + (u0,))
            def _p3(j, c3):
                u = c3[-1]
                out = []
                for k in range(C):
                    aa, ab, idxv = c3[k]
                    row = pl.ds(j * CL + k * L, L)
                    aa = aa + a_v[row]
                    ab = ab + b_v[row]
                    a_v[row] = zero16
                    b_v[row] = zero16
                    val = aa + u * ab
                    plsc.store_scatter(out_v, [idxv], val)
                    out.append((aa, ab, idxv + one16i))
                out.append(u + 1.0)
                return tuple(out)

            return carry2

        lax.fori_loop(0, G, group_body, 0)

    issue_in(0, bins0_v, w0_v, semi0)
    issue_out(0, out0_v, semo0)
    issue_out(1, out1_v, semo1)

    def pair_body(p, carry):
        s0 = p * 2
        wait_in(bins0_v, w0_v, semi0)
        issue_in(jnp.minimum(s0 + 1, NSUP - 1), bins1_v, w1_v, semi1)
        wait_out(out0_v, semo0)
        compute(bins0_v, w0_v, out0_v)
        issue_out(s0, out0_v, semo0)
        s1 = s0 + 1
        wait_in(bins1_v, w1_v, semi1)
        issue_in(jnp.minimum(s1 + 1, NSUP - 1), bins0_v, w0_v, semi0)
        wait_out(out1_v, semo1)
        compute(bins1_v, w1_v, out1_v)
        issue_out(s1, out1_v, semo1)
        return carry

    lax.fori_loop(0, NSUP // 2, pair_body, 0)

    wait_in(bins0_v, w0_v, semi0)
    wait_out(out0_v, semo0)
    wait_out(out1_v, semo1)


@jax.jit
def _run(bins, weights):
    mesh = plsc.VectorSubcoreMesh(core_axis_name="c", subcore_axis_name="s")
    kfn = pl.kernel(
        _sc_body,
        out_type=jax.ShapeDtypeStruct((N_R * NBP,), jnp.float32),
        mesh=mesh,
        compiler_params=pltpu.CompilerParams(needs_layout_passes=False),
        scratch_types=[
            pltpu.VMEM((SZ,), jnp.float32),
            pltpu.VMEM((SZ,), jnp.float32),
            pltpu.VMEM((SZ,), jnp.float32),
            pltpu.VMEM((SZ,), jnp.float32),
            pltpu.VMEM((SZ,), jnp.float32),
            pltpu.VMEM((SZ,), jnp.float32),
            pltpu.VMEM(((N_S + 1) * CL,), jnp.float32),
            pltpu.VMEM(((N_S + 1) * CL,), jnp.float32),
            pltpu.SemaphoreType.DMA,
            pltpu.SemaphoreType.DMA,
            pltpu.SemaphoreType.DMA,
            pltpu.SemaphoreType.DMA,
        ],
    )
    w_pad = jnp.pad(weights + 1e-5, ((0, 0), (0, 1)))
    out = kfn(bins.reshape(-1), w_pad.reshape(-1))
    return out.reshape(N_R, NBP)[:, :N_S]


def kernel(bins, weights, n_samples):
    return _run(bins, weights)


# speedup vs baseline: 1.0055x; 1.0055x over previous
import functools

import jax
import jax.numpy as jnp
from jax import lax
from jax.experimental import pallas as pl
from jax.experimental.pallas import tpu as pltpu
from jax.experimental.pallas import tpu_sc as plsc

N_R = 65536
N_B = 128
N_S = 128
L = 16
N_WORKERS = 32
NBP = N_B + 1
C = 2
CL = C * L
G = 4
SZ = G * CL * NBP
NSUP = N_R // N_WORKERS // (G * CL)


def _sc_body(bins_hbm, w_hbm, out_hbm,
             bins0_v, w0_v, out0_v, bins1_v, w1_v, out1_v,
             a_v, b_v, semi0, semi1, semo0, semo1):
    cid = lax.axis_index("c")
    sid = lax.axis_index("s")
    wid = sid * 2 + cid
    rays_per_w = N_R // N_WORKERS

    lane = lax.broadcasted_iota(jnp.int32, (L,), 0)
    laneP = [lane * NBP + k * (L * NBP) for k in range(C)]
    klane = [jnp.full((L,), k * L, jnp.int32) + lane for k in range(C)]
    zero16 = jnp.zeros((L,), jnp.float32)
    one16i = jnp.ones((L,), jnp.int32)
    MAGIC = 12582912.0

    def rbase(s):
        return (wid * rays_per_w + s * (G * CL)) * NBP

    def issue_in(s, bins_v, w_v, sem):
        pltpu.async_copy(bins_hbm.at[pl.ds(rbase(s), SZ)], bins_v, sem)
        pltpu.async_copy(w_hbm.at[pl.ds(rbase(s), SZ)], w_v, sem)

    def wait_in(bins_v, w_v, sem):
        pltpu.make_async_copy(bins_hbm.at[pl.ds(0, SZ)], bins_v, sem).wait()
        pltpu.make_async_copy(w_hbm.at[pl.ds(0, SZ)], w_v, sem).wait()

    def issue_out(s, out_v, sem):
        pltpu.async_copy(out_v, out_hbm.at[pl.ds(rbase(s), SZ)], sem)

    def wait_out(out_v, sem):
        pltpu.make_async_copy(out_v, out_hbm.at[pl.ds(0, SZ)], sem).wait()

    def _init(j, c):
        a_v[pl.ds(j * L, L)] = zero16
        b_v[pl.ds(j * L, L)] = zero16
        return c
    lax.fori_loop(0, N_S * C, _init, 0)

    def compute(bins_v, w_v, out_v):
        def group_body(g, carry2):
            ofs = g * (CL * NBP)
            gP = [laneP[k] + ofs for k in range(C)]

            @plsc.parallel_loop(0, N_B, unroll=4,
                                carry=tuple((zero16, gP[k]) for k in range(C)))
            def tot(i, c):
                out = []
                for k in range(C):
                    acc, idxv = c[k]
                    col = plsc.load_gather(w_v, [idxv])
                    out.append((acc + col, idxv + one16i))
                return tuple(out)

            inv128 = [128.0 / tot[k][0] for k in range(C)]

            def scatter(k, scdf0, dA, dB):
                jf = (scdf0 + MAGIC) - MAGIC
                j = jf.astype(jnp.int32)
                j = jnp.minimum(j, N_S)
                idx = j * CL + klane[k]
                plsc.addupdate_scatter(a_v, [idx], dA)
                plsc.addupdate_scatter(b_v, [idx], dB)

            bins0 = [plsc.load_gather(bins_v, [gP[k]]) for k in range(C)]

            @plsc.parallel_loop(0, N_B, unroll=4,
                                carry=tuple(
                                    (zero16, zero16, bins0[k], zero16, zero16,
                                     gP[k]) for k in range(C)))
            def cfin(i, c):
                out = []
                for k in range(C):
                    acc, scdf0, bi, A_prev, B_prev, idxv = c[k]
                    wcol = plsc.load_gather(w_v, [idxv])
                    acc = acc + wcol
                    scdf1 = acc * inv128[k]
                    idxv1 = idxv + one16i
                    bi1 = plsc.load_gather(bins_v, [idxv1])
                    denom = scdf1 - scdf0
                    denom = jnp.where(denom < 1.28e-3, 128.0, denom)
                    B = (bi1 - bi) / denom
                    A = bi - scdf0 * B
                    scatter(k, scdf0, A - A_prev, B - B_prev)
                    out.append((acc, scdf1, bi1, A, B, idxv1))
                return tuple(out)

            for k in range(C):
                _, scdfN, binsN, A_prev, B_prev, _ = cfin[k]
                scatter(k, scdfN, binsN - A_prev, -B_prev)

            u0 = jnp.full((L,), 0.5, jnp.float32)

            @plsc.parallel_loop(0, N_S, unroll=4,
                                carry=tuple((zero16, zero16, gP[k])
                                            for k in range(C)) + (u0,))
            def _p3(j, c3):
                u = c3[-1]
                out = []
                for k in range(C):
                    aa, ab, idxv = c3[k]
                    row = pl.ds(j * CL + k * L, L)
                    aa = aa + a_v[row]
                    ab = ab + b_v[row]
                    a_v[row] = zero16
                    b_v[row] = zero16
                    val = aa + u * ab
                    plsc.store_scatter(out_v, [idxv], val)
                    out.append((aa, ab, idxv + one16i))
                out.append(u + 1.0)
                return tuple(out)

            return carry2

        lax.fori_loop(0, G, group_body, 0)

    issue_in(0, bins0_v, w0_v, semi0)
    issue_out(0, out0_v, semo0)
    issue_out(1, out1_v, semo1)

    def pair_body(p, carry):
        s0 = p * 2
        wait_in(bins0_v, w0_v, semi0)
        issue_in(jnp.minimum(s0 + 1, NSUP - 1), bins1_v, w1_v, semi1)
        wait_out(out0_v, semo0)
        compute(bins0_v, w0_v, out0_v)
        issue_out(s0, out0_v, semo0)
        s1 = s0 + 1
        wait_in(bins1_v, w1_v, semi1)
        issue_in(jnp.minimum(s1 + 1, NSUP - 1), bins0_v, w0_v, semi0)
        wait_out(out1_v, semo1)
        compute(bins1_v, w1_v, out1_v)
        issue_out(s1, out1_v, semo1)
        return carry

    lax.fori_loop(0, NSUP // 2, pair_body, 0)

    wait_in(bins0_v, w0_v, semi0)
    wait_out(out0_v, semo0)
    wait_out(out1_v, semo1)


@jax.jit
def _run(bins, weights):
    mesh = plsc.VectorSubcoreMesh(core_axis_name="c", subcore_axis_name="s")
    kfn = pl.kernel(
        _sc_body,
        out_type=jax.ShapeDtypeStruct((N_R * NBP,), jnp.float32),
        mesh=mesh,
        compiler_params=pltpu.CompilerParams(needs_layout_passes=False),
        scratch_types=[
            pltpu.VMEM((SZ,), jnp.float32),
            pltpu.VMEM((SZ,), jnp.float32),
            pltpu.VMEM((SZ,), jnp.float32),
            pltpu.VMEM((SZ,), jnp.float32),
            pltpu.VMEM((SZ,), jnp.float32),
            pltpu.VMEM((SZ,), jnp.float32),
            pltpu.VMEM(((N_S + 1) * CL,), jnp.float32),
            pltpu.VMEM(((N_S + 1) * CL,), jnp.float32),
            pltpu.SemaphoreType.DMA,
            pltpu.SemaphoreType.DMA,
            pltpu.SemaphoreType.DMA,
            pltpu.SemaphoreType.DMA,
        ],
    )
    w_pad = jnp.pad(weights + 1e-5, ((0, 0), (0, 1)))
    out = kfn(bins.reshape(-1), w_pad.reshape(-1))
    return out.reshape(N_R, NBP)[:, :N_S]


def kernel(bins, weights, n_samples):
    return _run(bins, weights)
